# baseline (speedup 1.0000x reference)
import functools

import jax
import jax.numpy as jnp
from jax import lax
from jax.experimental import pallas as pl
from jax.experimental.pallas import tpu as pltpu
from jax.experimental.pallas import tpu_sc as plsc

NUM_USERS = 1000000
NUM_ITEMS = 100000
EMB_SIZE = 32
BATCH = 16384

_info = plsc.get_sparse_core_info()
_NC, _NS, _L = _info.num_cores, _info.num_subcores, _info.num_lanes
_NW = _NC * _NS
_BPW = BATCH // _NW


def _body(user_hbm, item_hbm, uemb_hbm, iemb_hbm, ubias_hbm, ibias_hbm,
          out_hbm, uidx_v, iidx_v, urows_v, irows_v, ub_v, ib_v, out_v,
          sem_u, sem_i, sem_ub, sem_ib):
    wid = lax.axis_index("s") * _NC + lax.axis_index("c")
    base = wid * _BPW

    pltpu.sync_copy(user_hbm.at[pl.ds(base, _BPW)], uidx_v)
    pltpu.sync_copy(item_hbm.at[pl.ds(base, _BPW)], iidx_v)

    cp_u = pltpu.async_copy(uemb_hbm.at[uidx_v], urows_v, sem_u)
    cp_i = pltpu.async_copy(iemb_hbm.at[iidx_v], irows_v, sem_i)
    cp_ub = pltpu.async_copy(ubias_hbm.at[uidx_v], ub_v, sem_ub)
    cp_ib = pltpu.async_copy(ibias_hbm.at[iidx_v], ib_v, sem_ib)
    cp_u.wait()
    cp_i.wait()
    cp_ub.wait()
    cp_ib.wait()

    def group(g, carry):
        rows = g * _L + lax.iota(jnp.int32, _L)
        acc = ub_v[pl.ds(g * _L, _L)] + ib_v[pl.ds(g * _L, _L)]
        for e in range(EMB_SIZE):
            col = jnp.full((_L,), e, jnp.int32)
            u = plsc.load_gather(urows_v, [rows, col])
            i = plsc.load_gather(irows_v, [rows, col])
            acc = acc + u * i
        out_v[pl.ds(g * _L, _L)] = acc
        return carry

    lax.fori_loop(0, _BPW // _L, group, 0)

    pltpu.sync_copy(out_v, out_hbm.at[pl.ds(base, _BPW)])


@jax.jit
def kernel(user, item, user_embedding, item_embedding, user_bias, item_bias):
    mesh = plsc.VectorSubcoreMesh(core_axis_name="c", subcore_axis_name="s")
    run = functools.partial(
        pl.kernel,
        out_type=jax.ShapeDtypeStruct((BATCH,), jnp.float32),
        mesh=mesh,
        compiler_params=pltpu.CompilerParams(
            needs_layout_passes=False, use_tc_tiling_on_sc=False),
        scratch_types=[
            pltpu.VMEM((_BPW,), jnp.int32),
            pltpu.VMEM((_BPW,), jnp.int32),
            pltpu.VMEM((_BPW, EMB_SIZE), jnp.float32),
            pltpu.VMEM((_BPW, EMB_SIZE), jnp.float32),
            pltpu.VMEM((_BPW,), jnp.float32),
            pltpu.VMEM((_BPW,), jnp.float32),
            pltpu.VMEM((_BPW,), jnp.float32),
            pltpu.SemaphoreType.DMA,
            pltpu.SemaphoreType.DMA,
            pltpu.SemaphoreType.DMA,
            pltpu.SemaphoreType.DMA,
        ],
    )(_body)
    return run(user.astype(jnp.int32), item.astype(jnp.int32),
               user_embedding, item_embedding,
               user_bias.reshape(-1), item_bias.reshape(-1))


# speedup vs baseline: 1.0030x; 1.0030x over previous
import functools

import jax
import jax.numpy as jnp
from jax import lax
from jax.experimental import pallas as pl
from jax.experimental.pallas import tpu as pltpu
from jax.experimental.pallas import tpu_sc as plsc

NUM_USERS = 1000000
NUM_ITEMS = 100000
EMB_SIZE = 32
BATCH = 16384

_info = plsc.get_sparse_core_info()
_NC, _NS, _L = _info.num_cores, _info.num_subcores, _info.num_lanes
_NW = _NC * _NS
_BPW = BATCH // _NW


def _body(user_hbm, item_hbm, uemb_hbm, iemb_hbm, ubias_hbm, ibias_hbm,
          out_hbm, uidx_v, iidx_v, udiv_v, idiv_v, urows_v, irows_v,
          ub_v, ib_v, out_v, sem_u, sem_i, sem_ub, sem_ib):
    wid = lax.axis_index("s") * _NC + lax.axis_index("c")
    base = wid * _BPW

    pltpu.sync_copy(user_hbm.at[pl.ds(base, _BPW)], uidx_v)
    pltpu.sync_copy(item_hbm.at[pl.ds(base, _BPW)], iidx_v)

    cp_u = pltpu.async_copy(uemb_hbm.at[uidx_v], urows_v, sem_u)
    cp_i = pltpu.async_copy(iemb_hbm.at[iidx_v], irows_v, sem_i)

    def shift(g, carry):
        sl = pl.ds(g * _L, _L)
        udiv_v[sl] = lax.shift_right_logical(uidx_v[sl], 3)
        idiv_v[sl] = lax.shift_right_logical(iidx_v[sl], 3)
        return carry

    lax.fori_loop(0, _BPW // _L, shift, 0)

    cp_ub = pltpu.async_copy(ubias_hbm.at[udiv_v], ub_v, sem_ub)
    cp_ib = pltpu.async_copy(ibias_hbm.at[idiv_v], ib_v, sem_ib)
    cp_u.wait()
    cp_i.wait()
    cp_ub.wait()
    cp_ib.wait()

    seven = jnp.full((_L,), 7, jnp.int32)

    def group(g, carry):
        sl = pl.ds(g * _L, _L)
        rows = g * _L + lax.iota(jnp.int32, _L)
        acc = plsc.load_gather(ub_v, [rows, uidx_v[sl] & seven])
        acc = acc + plsc.load_gather(ib_v, [rows, iidx_v[sl] & seven])
        for e in range(EMB_SIZE):
            col = jnp.full((_L,), e, jnp.int32)
            u = plsc.load_gather(urows_v, [rows, col])
            i = plsc.load_gather(irows_v, [rows, col])
            acc = acc + u * i
        out_v[sl] = acc
        return carry

    lax.fori_loop(0, _BPW // _L, group, 0)

    pltpu.sync_copy(out_v, out_hbm.at[pl.ds(base, _BPW)])


@jax.jit
def kernel(user, item, user_embedding, item_embedding, user_bias, item_bias):
    mesh = plsc.VectorSubcoreMesh(core_axis_name="c", subcore_axis_name="s")
    run = functools.partial(
        pl.kernel,
        out_type=jax.ShapeDtypeStruct((BATCH,), jnp.float32),
        mesh=mesh,
        compiler_params=pltpu.CompilerParams(
            needs_layout_passes=False, use_tc_tiling_on_sc=False),
        scratch_types=[
            pltpu.VMEM((_BPW,), jnp.int32),
            pltpu.VMEM((_BPW,), jnp.int32),
            pltpu.VMEM((_BPW,), jnp.int32),
            pltpu.VMEM((_BPW,), jnp.int32),
            pltpu.VMEM((_BPW, EMB_SIZE), jnp.float32),
            pltpu.VMEM((_BPW, EMB_SIZE), jnp.float32),
            pltpu.VMEM((_BPW, 8), jnp.float32),
            pltpu.VMEM((_BPW, 8), jnp.float32),
            pltpu.VMEM((_BPW,), jnp.float32),
            pltpu.SemaphoreType.DMA,
            pltpu.SemaphoreType.DMA,
            pltpu.SemaphoreType.DMA,
            pltpu.SemaphoreType.DMA,
        ],
    )(_body)
    return run(user, item, user_embedding, item_embedding,
               user_bias.reshape(-1, 8), item_bias.reshape(-1, 8))
